# Initial kernel scaffold; baseline (speedup 1.0000x reference)
import jax
import jax.numpy as jnp
from jax.experimental import pallas as pl


def kernel(z_h, edge_index_h_h, We1, be1, We2, be2, Ww1, bw1, Ww2, bw2, Wn1, bn1, Wn2, bn2):
    raise NotImplementedError("write your pallas kernel here")


# speedup vs baseline: 5.7056x; 5.7056x over previous
import functools

import jax
import jax.numpy as jnp
from jax import lax
from jax.experimental import pallas as pl
from jax.experimental.pallas import tpu as pltpu
from jax.experimental.pallas import tpu_sc as plsc

_B, _N, _F, _E, _H = 4, 2500, 13, 320000, 128
_BN = _B * _N
_FP = 16

_NC, _NS = 2, 16
_NW = _NC * _NS
_EB = 128
_NBATCH = 80
_GRP = 8
_PER_W = _NBATCH * _EB
_E_PAD = _NW * _PER_W
_ROWS_PER_TILE = _BN // _NS

_BLK_E = 2048
_BLK_N = 2000

@functools.lru_cache(maxsize=None)
def _sc_mesh():
    return plsc.VectorSubcoreMesh(
        core_axis_name="c", subcore_axis_name="s",
        num_cores=_NC, num_subcores=_NS)


def _gather_body(z_hbm, src_hbm, tgt_hbm, zs_out, zt_out, teff_out,
                 idx_s, idx_t, rows_s, rows_t, sem_s, sem_t):
    c = lax.axis_index("c")
    s = lax.axis_index("s")
    wid = s * _NC + c
    base = wid * _PER_W

    pltpu.sync_copy(src_hbm.at[wid], idx_s)
    pltpu.sync_copy(tgt_hbm.at[wid], idx_t)

    def _outer(k, carry):
        def _inner(l, carry2):
            sv = idx_s[k, pl.ds(l * 16, 16)]
            tv = idx_t[k, pl.ds(l * 16, 16)]
            idx_t[k, pl.ds(l * 16, 16)] = sv - lax.rem(sv, _N) + lax.rem(tv, _N)
            return carry2
        return lax.fori_loop(0, _EB // 16, _inner, carry)
    lax.fori_loop(0, _NBATCH, _outer, 0)

    pltpu.sync_copy(idx_t, teff_out.at[wid])

    def _gbody(g, carry):
        descs = []
        for j in range(_GRP):
            b = g * _GRP + j
            descs.append(pltpu.async_copy(
                z_hbm.at[idx_s.at[b]], rows_s.at[pl.ds(j * _EB, _EB)], sem_s))
            descs.append(pltpu.async_copy(
                z_hbm.at[idx_t.at[b]], rows_t.at[pl.ds(j * _EB, _EB)], sem_t))
        for d in descs:
            d.wait()
        row0 = base + g * (_GRP * _EB)
        pltpu.sync_copy(rows_s, zs_out.at[pl.ds(row0, _GRP * _EB)])
        pltpu.sync_copy(rows_t, zt_out.at[pl.ds(row0, _GRP * _EB)])
        return carry
    lax.fori_loop(0, _NBATCH // _GRP, _gbody, 0)


@functools.lru_cache(maxsize=None)
def _gather_call():
    return pl.kernel(
        _gather_body,
        out_type=(
            jax.ShapeDtypeStruct((_E_PAD, _FP), jnp.float32),
            jax.ShapeDtypeStruct((_E_PAD, _FP), jnp.float32),
            jax.ShapeDtypeStruct((_NW, _NBATCH, _EB), jnp.int32),
        ),
        mesh=_sc_mesh(),
        scratch_types=(
            pltpu.VMEM((_NBATCH, _EB), jnp.int32),
            pltpu.VMEM((_NBATCH, _EB), jnp.int32),
            pltpu.VMEM((_GRP * _EB, _FP), jnp.float32),
            pltpu.VMEM((_GRP * _EB, _FP), jnp.float32),
            pltpu.SemaphoreType.DMA,
            pltpu.SemaphoreType.DMA,
        ),
        compiler_params=pltpu.CompilerParams(use_tc_tiling_on_sc=False),
    )


def _scatter_body(wm_hbm, teff_hbm, zeros_hbm, out_hbm, idx_t, rows, acc, sem):
    c = lax.axis_index("c")
    s = lax.axis_index("s")
    wid = s * _NC + c
    r0 = s * _ROWS_PER_TILE

    pltpu.sync_copy(zeros_hbm.at[pl.ds(r0, _ROWS_PER_TILE)],
                    acc.at[pl.ds(r0, _ROWS_PER_TILE)])
    pltpu.sync_copy(teff_hbm.at[wid], idx_t)
    plsc.subcore_barrier()

    def _gbody(g, carry):
        pltpu.sync_copy(
            wm_hbm.at[pl.ds(wid * _PER_W + g * (_GRP * _EB), _GRP * _EB)], rows)
        descs = []
        for j in range(_GRP):
            descs.append(pltpu.async_copy(
                rows.at[pl.ds(j * _EB, _EB)], acc.at[idx_t.at[g * _GRP + j]],
                sem, add=True))
        for d in descs:
            d.wait()
        return carry
    lax.fori_loop(0, _NBATCH // _GRP, _gbody, 0)

    plsc.subcore_barrier()
    pltpu.sync_copy(acc.at[pl.ds(r0, _ROWS_PER_TILE)],
                    out_hbm.at[pl.ds(c * _BN + r0, _ROWS_PER_TILE)])


@functools.lru_cache(maxsize=None)
def _scatter_call():
    return pl.kernel(
        _scatter_body,
        out_type=jax.ShapeDtypeStruct((_NC * _BN, _FP), jnp.float32),
        mesh=_sc_mesh(),
        scratch_types=(
            pltpu.VMEM((_NBATCH, _EB), jnp.int32),
            pltpu.VMEM((_GRP * _EB, _FP), jnp.float32),
            pltpu.VMEM_SHARED((_BN, _FP), jnp.float32),
            pltpu.SemaphoreType.DMA,
        ),
        compiler_params=pltpu.CompilerParams(use_tc_tiling_on_sc=False),
    )


def _edge_body(zs_ref, zt_ref, w1_ref, b1_ref, w2_ref, b2_ref, wm_ref):
    zs = zs_ref[...]
    zt = zt_ref[...]
    d = zs[:, 0:3] - zt[:, 0:3]
    dist = jnp.sum(d * d, axis=1, keepdims=True)
    a1, a2, a3 = zs[:, 3:4], zs[:, 4:5], zs[:, 5:6]
    t1, t2, t3 = zt[:, 3:4], zt[:, 4:5], zt[:, 5:6]
    cx = a2 * t3 - a3 * t2
    cy = a3 * t1 - a1 * t3
    cz = a1 * t2 - a2 * t1
    absc = jnp.sqrt(cx * cx + cy * cy + cz * cz)
    x = jnp.concatenate(
        [zs[:, 0:13], zt[:, 0:13], d, dist, cx, cy, cz, absc], axis=1)
    h = jnp.tanh(
        jnp.dot(x, w1_ref[...], preferred_element_type=jnp.float32) + b1_ref[...])
    y = jnp.dot(h, w2_ref[...], preferred_element_type=jnp.float32) + b2_ref[...]
    w = jax.nn.sigmoid(y[:, 13:14])
    col = lax.broadcasted_iota(jnp.int32, y.shape, 1)
    row = lax.broadcasted_iota(jnp.int32, y.shape, 0) + pl.program_id(0) * _BLK_E
    wm_ref[...] = jnp.where((col < _F) & (row < _E), y * w, 0.0)


def _edge_call(zs, zt, w1, b1, w2, b2):
    grid = _E_PAD // _BLK_E
    return pl.pallas_call(
        _edge_body,
        grid=(grid,),
        in_specs=[
            pl.BlockSpec((_BLK_E, _FP), lambda i: (i, 0)),
            pl.BlockSpec((_BLK_E, _FP), lambda i: (i, 0)),
            pl.BlockSpec((2 * _F + 8, 2 * _H), lambda i: (0, 0)),
            pl.BlockSpec((1, 2 * _H), lambda i: (0, 0)),
            pl.BlockSpec((2 * _H, _FP), lambda i: (0, 0)),
            pl.BlockSpec((1, _FP), lambda i: (0, 0)),
        ],
        out_specs=pl.BlockSpec((_BLK_E, _FP), lambda i: (i, 0)),
        out_shape=jax.ShapeDtypeStruct((_E_PAD, _FP), jnp.float32),
        compiler_params=pltpu.CompilerParams(
            dimension_semantics=("arbitrary",)),
    )(zs, zt, w1, b1, w2, b2)


def _node_body(z_ref, m_ref, w1_ref, b1_ref, w2_ref, b2_ref, out_ref):
    z = z_ref[...]
    m = m_ref[0] + m_ref[1]
    x = jnp.concatenate([z[:, 0:13], m[:, 0:13]], axis=1)
    h = jnp.tanh(
        jnp.dot(x, w1_ref[...], preferred_element_type=jnp.float32) + b1_ref[...])
    out_ref[...] = (
        jnp.dot(h, w2_ref[...], preferred_element_type=jnp.float32) + b2_ref[...])


def _node_call(z_pad, m_parts, w1, b1, w2, b2):
    grid = _BN // _BLK_N
    return pl.pallas_call(
        _node_body,
        grid=(grid,),
        in_specs=[
            pl.BlockSpec((_BLK_N, _FP), lambda i: (i, 0)),
            pl.BlockSpec((_NC, _BLK_N, _FP), lambda i: (0, i, 0)),
            pl.BlockSpec((2 * _F, _H), lambda i: (0, 0)),
            pl.BlockSpec((1, _H), lambda i: (0, 0)),
            pl.BlockSpec((_H, _F), lambda i: (0, 0)),
            pl.BlockSpec((1, _F), lambda i: (0, 0)),
        ],
        out_specs=pl.BlockSpec((_BLK_N, _F), lambda i: (i, 0)),
        out_shape=jax.ShapeDtypeStruct((_BN, _F), jnp.float32),
        compiler_params=pltpu.CompilerParams(
            dimension_semantics=("arbitrary",)),
    )(z_pad, m_parts, w1, b1, w2, b2)


def kernel(z_h, edge_index_h_h, We1, be1, We2, be2, Ww1, bw1, Ww2, bw2,
           Wn1, bn1, Wn2, bn2):
    z_flat = z_h.reshape(_BN, _F)
    z_pad = jnp.pad(z_flat, ((0, 0), (0, _FP - _F)))

    src = edge_index_h_h[0].astype(jnp.int32)
    tgt = edge_index_h_h[1].astype(jnp.int32)
    src_p = jnp.pad(src, (0, _E_PAD - _E)).reshape(_NW, _NBATCH, _EB)
    tgt_p = jnp.pad(tgt, (0, _E_PAD - _E)).reshape(_NW, _NBATCH, _EB)

    zs, zt, teff = _gather_call()(z_pad, src_p, tgt_p)

    w1c = jnp.concatenate([We1, Ww1], axis=1)
    b1c = jnp.concatenate([be1, bw1])[None, :]
    w2c = jnp.zeros((2 * _H, _FP), jnp.float32)
    w2c = w2c.at[0:_H, 0:_F].set(We2)
    w2c = w2c.at[_H:2 * _H, _F:_F + 1].set(Ww2)
    b2c = jnp.zeros((_FP,), jnp.float32)
    b2c = b2c.at[0:_F].set(be2)
    b2c = b2c.at[_F].set(bw2[0])
    b2c = b2c[None, :]

    wm = _edge_call(zs, zt, w1c, b1c, w2c, b2c)

    zeros_acc = jnp.zeros((_BN, _FP), jnp.float32)
    m_parts = _scatter_call()(wm, teff, zeros_acc)

    delta = _node_call(z_pad, m_parts.reshape(_NC, _BN, _FP),
                       Wn1, bn1[None, :], Wn2, bn2[None, :])
    return delta.reshape(_B, _N, _F)


# speedup vs baseline: 13.3051x; 2.3320x over previous
import functools

import jax
import jax.numpy as jnp
from jax import lax
from jax.experimental import pallas as pl
from jax.experimental.pallas import tpu as pltpu
from jax.experimental.pallas import tpu_sc as plsc

_B, _N, _F, _E, _H = 4, 2500, 13, 320000, 128
_BN = _B * _N
_FP = 16

_NC, _NS = 2, 16
_NW = _NC * _NS
_EB = 128
_NBATCH = 80
_GRP = 4
_E_PAD = _NW * _NBATCH * _EB
_NBT = _NW * _NBATCH
_NB0 = 104
_NB1 = 56
_ROWS_PER_TILE = _BN // _NS

_BLK_E = 4096
_BLK_N = 2000

@functools.lru_cache(maxsize=None)
def _sc_mesh():
    return plsc.VectorSubcoreMesh(
        core_axis_name="c", subcore_axis_name="s",
        num_cores=_NC, num_subcores=_NS)


def _teff_body(src_ref, tgt_ref, out_ref):
    s = src_ref[...]
    t = tgt_ref[...]
    out_ref[...] = s - lax.rem(s, _N) + lax.rem(t, _N)


def _teff_call(src_p, tgt_p):
    n_rows = _NW * _NBATCH
    return pl.pallas_call(
        _teff_body,
        grid=(1,),
        in_specs=[
            pl.BlockSpec((n_rows, _EB), lambda i: (0, 0)),
            pl.BlockSpec((n_rows, _EB), lambda i: (0, 0)),
        ],
        out_specs=pl.BlockSpec((n_rows, _EB), lambda i: (0, 0)),
        out_shape=jax.ShapeDtypeStruct((n_rows, _EB), jnp.int32),
    )(src_p, tgt_p)


def _gather_pipe(z_hbm, src_hbm, teff_hbm, zs_out, zt_out,
                 idx_s, idx_t, rows_s, rows_t, sems_g, sems_o, b0, nb):
    pltpu.sync_copy(src_hbm.at[pl.ds(b0, nb)], idx_s.at[pl.ds(0, nb)])
    pltpu.sync_copy(teff_hbm.at[pl.ds(b0, nb)], idx_t.at[pl.ds(0, nb)])
    base = b0 * _EB

    grows = _GRP * _EB
    rows_bufs = (rows_s, rows_t)
    outs = (zs_out, zt_out)

    def _fire(g, p):
        descs = []
        for j in range(_GRP):
            b = g * _GRP + j
            for idx, rows, sem in ((idx_s, rows_bufs[0], sems_g[p][0]),
                                   (idx_t, rows_bufs[1], sems_g[p][1])):
                descs.append(pltpu.async_copy(
                    z_hbm.at[idx.at[b]],
                    rows.at[pl.ds((p * _GRP + j) * _EB, _EB)], sem))
        return descs

    def _drain_out(p):
        for rows, out in zip(rows_bufs, outs):
            pltpu.make_async_copy(
                rows.at[pl.ds(p * grows, grows)],
                out.at[pl.ds(base, grows)], sems_o[p]).wait()

    def _pair(i, carry):
        descs = []
        for p in (0, 1):
            @pl.when(i > 0)
            def _():
                _drain_out(p)
            descs.append(_fire(2 * i + p, p))
        for p in (0, 1):
            for d in descs[p]:
                d.wait()
            row0 = base + (2 * i + p) * grows
            for rows, out in zip(rows_bufs, outs):
                pltpu.async_copy(rows.at[pl.ds(p * grows, grows)],
                                 out.at[pl.ds(row0, grows)], sems_o[p])
        return carry
    lax.fori_loop(0, nb // (2 * _GRP), _pair, 0)
    _drain_out(0)
    _drain_out(1)


def _gather_body(z_hbm, src_hbm, teff_hbm, zs_out, zt_out,
                 idx_s, idx_t, rows_s, rows_t,
                 sem_s0, sem_t0, sem_s1, sem_t1, sem_o0, sem_o1):
    c = lax.axis_index("c")
    s = lax.axis_index("s")
    sems_g = ((sem_s0, sem_t0), (sem_s1, sem_t1))
    sems_o = (sem_o0, sem_o1)

    @pl.when(c == 0)
    def _():
        _gather_pipe(z_hbm, src_hbm, teff_hbm, zs_out, zt_out,
                     idx_s, idx_t, rows_s, rows_t, sems_g, sems_o,
                     s * _NB0, _NB0)

    @pl.when(c == 1)
    def _():
        _gather_pipe(z_hbm, src_hbm, teff_hbm, zs_out, zt_out,
                     idx_s, idx_t, rows_s, rows_t, sems_g, sems_o,
                     _NS * _NB0 + s * _NB1, _NB1)


@functools.lru_cache(maxsize=None)
def _gather_call():
    return pl.kernel(
        _gather_body,
        out_type=(
            jax.ShapeDtypeStruct((_E_PAD, _FP), jnp.float32),
            jax.ShapeDtypeStruct((_E_PAD, _FP), jnp.float32),
        ),
        mesh=_sc_mesh(),
        scratch_types=(
            pltpu.VMEM((_NB0, _EB), jnp.int32),
            pltpu.VMEM((_NB0, _EB), jnp.int32),
            pltpu.VMEM((2 * _GRP * _EB, _FP), jnp.float32),
            pltpu.VMEM((2 * _GRP * _EB, _FP), jnp.float32),
            pltpu.SemaphoreType.DMA,
            pltpu.SemaphoreType.DMA,
            pltpu.SemaphoreType.DMA,
            pltpu.SemaphoreType.DMA,
            pltpu.SemaphoreType.DMA,
            pltpu.SemaphoreType.DMA,
        ),
        compiler_params=pltpu.CompilerParams(use_tc_tiling_on_sc=False),
    )


def _scatter_pipe(wm_hbm, teff_hbm, idx_t, rows, acc, sem, b0, nb):
    pltpu.sync_copy(teff_hbm.at[pl.ds(b0, nb)], idx_t.at[pl.ds(0, nb)])

    def _gbody(g, carry):
        pltpu.sync_copy(
            wm_hbm.at[pl.ds(b0 * _EB + g * (_GRP * _EB), _GRP * _EB)], rows)
        descs = []
        for j in range(_GRP):
            descs.append(pltpu.async_copy(
                rows.at[pl.ds(j * _EB, _EB)], acc.at[idx_t.at[g * _GRP + j]],
                sem, add=True))
        for d in descs:
            d.wait()
        return carry
    lax.fori_loop(0, nb // _GRP, _gbody, 0)


def _scatter_body(wm_hbm, teff_hbm, zeros_hbm, out_hbm, idx_t, rows, acc, sem):
    c = lax.axis_index("c")
    s = lax.axis_index("s")
    r0 = s * _ROWS_PER_TILE

    pltpu.sync_copy(zeros_hbm.at[pl.ds(r0, _ROWS_PER_TILE)],
                    acc.at[pl.ds(r0, _ROWS_PER_TILE)])
    plsc.subcore_barrier()

    @pl.when(c == 0)
    def _():
        _scatter_pipe(wm_hbm, teff_hbm, idx_t, rows, acc, sem,
                      s * _NB0, _NB0)

    @pl.when(c == 1)
    def _():
        _scatter_pipe(wm_hbm, teff_hbm, idx_t, rows, acc, sem,
                      _NS * _NB0 + s * _NB1, _NB1)

    plsc.subcore_barrier()
    pltpu.sync_copy(acc.at[pl.ds(r0, _ROWS_PER_TILE)],
                    out_hbm.at[pl.ds(c * _BN + r0, _ROWS_PER_TILE)])


@functools.lru_cache(maxsize=None)
def _scatter_call():
    return pl.kernel(
        _scatter_body,
        out_type=jax.ShapeDtypeStruct((_NC * _BN, _FP), jnp.float32),
        mesh=_sc_mesh(),
        scratch_types=(
            pltpu.VMEM((_NB0, _EB), jnp.int32),
            pltpu.VMEM((_GRP * _EB, _FP), jnp.float32),
            pltpu.VMEM_SHARED((_BN, _FP), jnp.float32),
            pltpu.SemaphoreType.DMA,
        ),
        compiler_params=pltpu.CompilerParams(use_tc_tiling_on_sc=False),
    )


def _edge_body(zs_ref, zt_ref, w1_ref, b1_ref, w2_ref, b2_ref, m_ref,
               esel_ref, p1_ref, p2_ref, wm_ref):
    zp = zs_ref[...]
    zq = zt_ref[...]
    zs = jnp.concatenate([zp[:, k * _FP:(k + 1) * _FP] for k in range(8)],
                         axis=0)
    zt = jnp.concatenate([zq[:, k * _FP:(k + 1) * _FP] for k in range(8)],
                         axis=0)
    col = lax.broadcasted_iota(jnp.int32, zs.shape, 1)
    d = zs - zt
    dsq = d * d
    zz = jnp.concatenate([zs, zt], axis=1)
    rots = jnp.dot(zz, p1_ref[...], preferred_element_type=jnp.float32)
    prod = rots[:, 0:2 * _FP] * rots[:, 2 * _FP:4 * _FP]
    cross = jnp.dot(prod, p2_ref[...], preferred_element_type=jnp.float32)
    csq = cross * cross
    s2 = jnp.dot(csq, m_ref[...], preferred_element_type=jnp.float32)
    absc = jnp.sqrt(s2)
    quad = jnp.where(col == 6, absc, jnp.where(col < 3, dsq, 0.0))
    x = jnp.concatenate([zz, cross, quad], axis=1)
    h = jnp.tanh(
        jnp.dot(x, w1_ref[...], preferred_element_type=jnp.float32) + b1_ref[...])
    y = jnp.dot(h, w2_ref[...], preferred_element_type=jnp.float32) + b2_ref[...]
    w = jax.nn.sigmoid(
        jnp.dot(y, esel_ref[...], preferred_element_type=jnp.float32))
    p_row = lax.broadcasted_iota(jnp.int32, y.shape, 0)
    rows8 = _BLK_E // 8
    e_loc = 8 * (p_row % rows8) + p_row // rows8
    row = e_loc + pl.program_id(0) * _BLK_E
    wm = jnp.where((col < _F) & (row < _E), y * w, 0.0)
    wm_ref[...] = jnp.concatenate(
        [wm[k * rows8:(k + 1) * rows8, :] for k in range(8)], axis=1)


def _edge_call(zs, zt, w1, b1, w2, b2, m_mat, esel, p1, p2):
    grid = _E_PAD // _BLK_E
    return pl.pallas_call(
        _edge_body,
        grid=(grid,),
        in_specs=[
            pl.BlockSpec((_BLK_E // 8, 8 * _FP), lambda i: (i, 0)),
            pl.BlockSpec((_BLK_E // 8, 8 * _FP), lambda i: (i, 0)),
            pl.BlockSpec((4 * _FP, 2 * _H), lambda i: (0, 0)),
            pl.BlockSpec((1, 2 * _H), lambda i: (0, 0)),
            pl.BlockSpec((2 * _H, _FP), lambda i: (0, 0)),
            pl.BlockSpec((1, _FP), lambda i: (0, 0)),
            pl.BlockSpec((_FP, _FP), lambda i: (0, 0)),
            pl.BlockSpec((_FP, _FP), lambda i: (0, 0)),
            pl.BlockSpec((2 * _FP, 4 * _FP), lambda i: (0, 0)),
            pl.BlockSpec((2 * _FP, _FP), lambda i: (0, 0)),
        ],
        out_specs=pl.BlockSpec((_BLK_E // 8, 8 * _FP), lambda i: (i, 0)),
        out_shape=jax.ShapeDtypeStruct((_E_PAD // 8, 8 * _FP), jnp.float32),
        compiler_params=pltpu.CompilerParams(
            dimension_semantics=("arbitrary",)),
    )(zs, zt, w1, b1, w2, b2, m_mat, esel, p1, p2)


def _node_body(z_ref, m_ref, w1_ref, b1_ref, w2_ref, b2_ref, out_ref):
    z = z_ref[...]
    m = m_ref[0] + m_ref[1]
    x = jnp.concatenate([z[:, 0:13], m[:, 0:13]], axis=1)
    h = jnp.tanh(
        jnp.dot(x, w1_ref[...], preferred_element_type=jnp.float32) + b1_ref[...])
    out_ref[...] = (
        jnp.dot(h, w2_ref[...], preferred_element_type=jnp.float32) + b2_ref[...])


def _node_call(z_pad, m_parts, w1, b1, w2, b2):
    grid = _BN // _BLK_N
    return pl.pallas_call(
        _node_body,
        grid=(grid,),
        in_specs=[
            pl.BlockSpec((_BLK_N, _FP), lambda i: (i, 0)),
            pl.BlockSpec((_NC, _BLK_N, _FP), lambda i: (0, i, 0)),
            pl.BlockSpec((2 * _F, _H), lambda i: (0, 0)),
            pl.BlockSpec((1, _H), lambda i: (0, 0)),
            pl.BlockSpec((_H, _F), lambda i: (0, 0)),
            pl.BlockSpec((1, _F), lambda i: (0, 0)),
        ],
        out_specs=pl.BlockSpec((_BLK_N, _F), lambda i: (i, 0)),
        out_shape=jax.ShapeDtypeStruct((_BN, _F), jnp.float32),
        compiler_params=pltpu.CompilerParams(
            dimension_semantics=("arbitrary",)),
    )(z_pad, m_parts, w1, b1, w2, b2)


def kernel(z_h, edge_index_h_h, We1, be1, We2, be2, Ww1, bw1, Ww2, bw2,
           Wn1, bn1, Wn2, bn2):
    z_flat = z_h.reshape(_BN, _F)
    z_pad = jnp.pad(z_flat, ((0, 0), (0, _FP - _F)))

    src = edge_index_h_h[0].astype(jnp.int32)
    tgt = edge_index_h_h[1].astype(jnp.int32)
    src2 = jnp.pad(src, (0, _E_PAD - _E)).reshape(_NW * _NBATCH, _EB)
    tgt2 = jnp.pad(tgt, (0, _E_PAD - _E)).reshape(_NW * _NBATCH, _EB)
    teff = _teff_call(src2, tgt2)

    zs, zt = _gather_call()(z_pad, src2, teff)

    w1c = jnp.concatenate([We1, Ww1], axis=1)
    b1c = jnp.concatenate([be1, bw1])[None, :]
    w64 = jnp.zeros((4 * _FP, 2 * _H), jnp.float32)
    w64 = w64.at[0:_F].set(w1c[0:13])
    w64 = w64.at[0:3].add(w1c[26:29])
    w64 = w64.at[_FP:_FP + _F].set(w1c[13:26])
    w64 = w64.at[_FP:_FP + 3].add(-w1c[26:29])
    w64 = w64.at[2 * _FP + 3:2 * _FP + 6].set(w1c[30:33])
    w64 = w64.at[3 * _FP + 0:3 * _FP + 3].set(
        jnp.broadcast_to(w1c[29], (3, 2 * _H)))
    w64 = w64.at[3 * _FP + 6].set(w1c[33])
    w2c = jnp.zeros((2 * _H, _FP), jnp.float32)
    w2c = w2c.at[0:_H, 0:_F].set(We2)
    w2c = w2c.at[_H:2 * _H, _F:_F + 1].set(Ww2)
    b2c = jnp.zeros((_FP,), jnp.float32)
    b2c = b2c.at[0:_F].set(be2)
    b2c = b2c.at[_F].set(bw2[0])
    b2c = b2c[None, :]
    m_mat = jnp.zeros((_FP, _FP), jnp.float32).at[3:6].set(1.0)
    esel = jnp.zeros((_FP, _FP), jnp.float32).at[_F].set(1.0)
    p1 = jnp.zeros((_FP, _FP), jnp.float32)
    p1 = p1.at[4, 3].set(1.0).at[5, 4].set(1.0).at[3, 5].set(1.0)
    p2 = jnp.zeros((_FP, _FP), jnp.float32)
    p2 = p2.at[5, 3].set(1.0).at[3, 4].set(1.0).at[4, 5].set(1.0)
    rot_big = jnp.zeros((2 * _FP, 4 * _FP), jnp.float32)
    rot_big = rot_big.at[0:_FP, 0:_FP].set(p1)
    rot_big = rot_big.at[0:_FP, _FP:2 * _FP].set(p2)
    rot_big = rot_big.at[_FP:2 * _FP, 2 * _FP:3 * _FP].set(p2)
    rot_big = rot_big.at[_FP:2 * _FP, 3 * _FP:4 * _FP].set(p1)
    eye = jnp.eye(_FP, dtype=jnp.float32)
    sub_big = jnp.concatenate([eye, -eye], axis=0)

    wm_p = _edge_call(zs.reshape(_E_PAD // 8, 8 * _FP),
                      zt.reshape(_E_PAD // 8, 8 * _FP),
                      w64, b1c, w2c, b2c, m_mat, esel, rot_big, sub_big)
    wm = wm_p.reshape(_E_PAD, _FP)

    zeros_acc = jnp.zeros((_BN, _FP), jnp.float32)
    m_parts = _scatter_call()(wm, teff, zeros_acc)

    delta = _node_call(z_pad, m_parts.reshape(_NC, _BN, _FP),
                       Wn1, bn1[None, :], Wn2, bn2[None, :])
    return delta.reshape(_B, _N, _F)


# speedup vs baseline: 13.5152x; 1.0158x over previous
import functools

import jax
import jax.numpy as jnp
from jax import lax
from jax.experimental import pallas as pl
from jax.experimental.pallas import tpu as pltpu
from jax.experimental.pallas import tpu_sc as plsc

_B, _N, _F, _E, _H = 4, 2500, 13, 320000, 128
_BN = _B * _N
_FP = 16

_NC, _NS = 2, 16
_NW = _NC * _NS
_EB = 128
_NBATCH = 80
_GRP = 4
_E_PAD = _NW * _NBATCH * _EB
_NBT = _NW * _NBATCH
_GNB0, _GNB1 = 128, 32
_SNB0, _SNB1 = 88, 72
_ROWS_PER_TILE = _BN // _NS

_BLK_E = 4096
_BLK_N = 2000

@functools.lru_cache(maxsize=None)
def _sc_mesh():
    return plsc.VectorSubcoreMesh(
        core_axis_name="c", subcore_axis_name="s",
        num_cores=_NC, num_subcores=_NS)


def _teff_body(src_ref, tgt_ref, out_ref):
    s = src_ref[...]
    t = tgt_ref[...]
    out_ref[...] = s - lax.rem(s, _N) + lax.rem(t, _N)


def _teff_call(src_p, tgt_p):
    n_rows = _NW * _NBATCH
    return pl.pallas_call(
        _teff_body,
        grid=(1,),
        in_specs=[
            pl.BlockSpec((n_rows, _EB), lambda i: (0, 0)),
            pl.BlockSpec((n_rows, _EB), lambda i: (0, 0)),
        ],
        out_specs=pl.BlockSpec((n_rows, _EB), lambda i: (0, 0)),
        out_shape=jax.ShapeDtypeStruct((n_rows, _EB), jnp.int32),
    )(src_p, tgt_p)


def _gather_pipe(z_hbm, src_hbm, teff_hbm, zs_out, zt_out,
                 idx_s, idx_t, rows_s, rows_t, sems_g, sems_o, b0, nb):
    pltpu.sync_copy(src_hbm.at[pl.ds(b0, nb)], idx_s.at[pl.ds(0, nb)])
    pltpu.sync_copy(teff_hbm.at[pl.ds(b0, nb)], idx_t.at[pl.ds(0, nb)])
    base = b0 * _EB

    grows = _GRP * _EB
    rows_bufs = (rows_s, rows_t)
    outs = (zs_out, zt_out)

    def _fire(g, p):
        descs = []
        for j in range(_GRP):
            b = g * _GRP + j
            for idx, rows, sem in ((idx_s, rows_bufs[0], sems_g[p][0]),
                                   (idx_t, rows_bufs[1], sems_g[p][1])):
                descs.append(pltpu.async_copy(
                    z_hbm.at[idx.at[b]],
                    rows.at[pl.ds((p * _GRP + j) * _EB, _EB)], sem))
        return descs

    def _drain_out(p):
        for rows, out in zip(rows_bufs, outs):
            pltpu.make_async_copy(
                rows.at[pl.ds(p * grows, grows)],
                out.at[pl.ds(base, grows)], sems_o[p]).wait()

    def _pair(i, carry):
        descs = []
        for p in (0, 1):
            @pl.when(i > 0)
            def _():
                _drain_out(p)
            descs.append(_fire(2 * i + p, p))
        for p in (0, 1):
            for d in descs[p]:
                d.wait()
            row0 = base + (2 * i + p) * grows
            for rows, out in zip(rows_bufs, outs):
                pltpu.async_copy(rows.at[pl.ds(p * grows, grows)],
                                 out.at[pl.ds(row0, grows)], sems_o[p])
        return carry
    lax.fori_loop(0, nb // (2 * _GRP), _pair, 0)
    _drain_out(0)
    _drain_out(1)


def _gather_body(z_hbm, src_hbm, teff_hbm, zs_out, zt_out,
                 idx_s, idx_t, rows_s, rows_t,
                 sem_s0, sem_t0, sem_s1, sem_t1, sem_o0, sem_o1):
    c = lax.axis_index("c")
    s = lax.axis_index("s")
    sems_g = ((sem_s0, sem_t0), (sem_s1, sem_t1))
    sems_o = (sem_o0, sem_o1)

    @pl.when(c == 0)
    def _():
        _gather_pipe(z_hbm, src_hbm, teff_hbm, zs_out, zt_out,
                     idx_s, idx_t, rows_s, rows_t, sems_g, sems_o,
                     s * _GNB0, _GNB0)

    @pl.when(c == 1)
    def _():
        _gather_pipe(z_hbm, src_hbm, teff_hbm, zs_out, zt_out,
                     idx_s, idx_t, rows_s, rows_t, sems_g, sems_o,
                     _NS * _GNB0 + s * _GNB1, _GNB1)


@functools.lru_cache(maxsize=None)
def _gather_call():
    return pl.kernel(
        _gather_body,
        out_type=(
            jax.ShapeDtypeStruct((_E_PAD, _FP), jnp.float32),
            jax.ShapeDtypeStruct((_E_PAD, _FP), jnp.float32),
        ),
        mesh=_sc_mesh(),
        scratch_types=(
            pltpu.VMEM((_GNB0, _EB), jnp.int32),
            pltpu.VMEM((_GNB0, _EB), jnp.int32),
            pltpu.VMEM((2 * _GRP * _EB, _FP), jnp.float32),
            pltpu.VMEM((2 * _GRP * _EB, _FP), jnp.float32),
            pltpu.SemaphoreType.DMA,
            pltpu.SemaphoreType.DMA,
            pltpu.SemaphoreType.DMA,
            pltpu.SemaphoreType.DMA,
            pltpu.SemaphoreType.DMA,
            pltpu.SemaphoreType.DMA,
        ),
        compiler_params=pltpu.CompilerParams(use_tc_tiling_on_sc=False),
    )


def _scatter_pipe(wm_hbm, teff_hbm, idx_t, rows, acc, sem, b0, nb):
    pltpu.sync_copy(teff_hbm.at[pl.ds(b0, nb)], idx_t.at[pl.ds(0, nb)])

    def _gbody(g, carry):
        pltpu.sync_copy(
            wm_hbm.at[pl.ds(b0 * _EB + g * (_GRP * _EB), _GRP * _EB)], rows)
        descs = []
        for j in range(_GRP):
            descs.append(pltpu.async_copy(
                rows.at[pl.ds(j * _EB, _EB)], acc.at[idx_t.at[g * _GRP + j]],
                sem, add=True))
        for d in descs:
            d.wait()
        return carry
    lax.fori_loop(0, nb // _GRP, _gbody, 0)


def _scatter_body(wm_hbm, teff_hbm, zeros_hbm, out_hbm, idx_t, rows, acc, sem):
    c = lax.axis_index("c")
    s = lax.axis_index("s")
    r0 = s * _ROWS_PER_TILE

    pltpu.sync_copy(zeros_hbm.at[pl.ds(r0, _ROWS_PER_TILE)],
                    acc.at[pl.ds(r0, _ROWS_PER_TILE)])
    plsc.subcore_barrier()

    @pl.when(c == 0)
    def _():
        _scatter_pipe(wm_hbm, teff_hbm, idx_t, rows, acc, sem,
                      s * _SNB0, _SNB0)

    @pl.when(c == 1)
    def _():
        _scatter_pipe(wm_hbm, teff_hbm, idx_t, rows, acc, sem,
                      _NS * _SNB0 + s * _SNB1, _SNB1)

    plsc.subcore_barrier()
    pltpu.sync_copy(acc.at[pl.ds(r0, _ROWS_PER_TILE)],
                    out_hbm.at[pl.ds(c * _BN + r0, _ROWS_PER_TILE)])


@functools.lru_cache(maxsize=None)
def _scatter_call():
    return pl.kernel(
        _scatter_body,
        out_type=jax.ShapeDtypeStruct((_NC * _BN, _FP), jnp.float32),
        mesh=_sc_mesh(),
        scratch_types=(
            pltpu.VMEM((_SNB0, _EB), jnp.int32),
            pltpu.VMEM((_GRP * _EB, _FP), jnp.float32),
            pltpu.VMEM_SHARED((_BN, _FP), jnp.float32),
            pltpu.SemaphoreType.DMA,
        ),
        compiler_params=pltpu.CompilerParams(use_tc_tiling_on_sc=False),
    )


def _edge_body(zs_ref, zt_ref, w1_ref, b1_ref, w2_ref, b2_ref, m_ref,
               esel_ref, p1_ref, p2_ref, wm_ref):
    zp = zs_ref[...]
    zq = zt_ref[...]
    zs = jnp.concatenate([zp[:, k * _FP:(k + 1) * _FP] for k in range(8)],
                         axis=0)
    zt = jnp.concatenate([zq[:, k * _FP:(k + 1) * _FP] for k in range(8)],
                         axis=0)
    col = lax.broadcasted_iota(jnp.int32, zs.shape, 1)
    d = zs - zt
    dsq = d * d
    zz = jnp.concatenate([zs, zt], axis=1)
    rots = jnp.dot(zz, p1_ref[...], preferred_element_type=jnp.float32)
    prod = rots[:, 0:2 * _FP] * rots[:, 2 * _FP:4 * _FP]
    cross = jnp.dot(prod, p2_ref[...], preferred_element_type=jnp.float32)
    csq = cross * cross
    s2 = jnp.dot(csq, m_ref[...], preferred_element_type=jnp.float32)
    absc = jnp.sqrt(s2)
    quad = jnp.where(col == 6, absc, jnp.where(col < 3, dsq, 0.0))
    x = jnp.concatenate([zz, cross, quad], axis=1)
    h = jnp.tanh(
        jnp.dot(x, w1_ref[...], preferred_element_type=jnp.float32) + b1_ref[...])
    y = jnp.dot(h, w2_ref[...], preferred_element_type=jnp.float32) + b2_ref[...]
    w = jax.nn.sigmoid(
        jnp.dot(y, esel_ref[...], preferred_element_type=jnp.float32))
    p_row = lax.broadcasted_iota(jnp.int32, y.shape, 0)
    rows8 = _BLK_E // 8
    e_loc = 8 * (p_row % rows8) + p_row // rows8
    row = e_loc + pl.program_id(0) * _BLK_E
    wm = jnp.where((col < _F) & (row < _E), y * w, 0.0)
    wm_ref[...] = jnp.concatenate(
        [wm[k * rows8:(k + 1) * rows8, :] for k in range(8)], axis=1)


def _edge_call(zs, zt, w1, b1, w2, b2, m_mat, esel, p1, p2):
    grid = _E_PAD // _BLK_E
    return pl.pallas_call(
        _edge_body,
        grid=(grid,),
        in_specs=[
            pl.BlockSpec((_BLK_E // 8, 8 * _FP), lambda i: (i, 0)),
            pl.BlockSpec((_BLK_E // 8, 8 * _FP), lambda i: (i, 0)),
            pl.BlockSpec((4 * _FP, 2 * _H), lambda i: (0, 0)),
            pl.BlockSpec((1, 2 * _H), lambda i: (0, 0)),
            pl.BlockSpec((2 * _H, _FP), lambda i: (0, 0)),
            pl.BlockSpec((1, _FP), lambda i: (0, 0)),
            pl.BlockSpec((_FP, _FP), lambda i: (0, 0)),
            pl.BlockSpec((_FP, _FP), lambda i: (0, 0)),
            pl.BlockSpec((2 * _FP, 4 * _FP), lambda i: (0, 0)),
            pl.BlockSpec((2 * _FP, _FP), lambda i: (0, 0)),
        ],
        out_specs=pl.BlockSpec((_BLK_E // 8, 8 * _FP), lambda i: (i, 0)),
        out_shape=jax.ShapeDtypeStruct((_E_PAD // 8, 8 * _FP), jnp.float32),
        compiler_params=pltpu.CompilerParams(
            dimension_semantics=("arbitrary",)),
    )(zs, zt, w1, b1, w2, b2, m_mat, esel, p1, p2)


def _node_body(z_ref, m_ref, w1_ref, b1_ref, w2_ref, b2_ref, out_ref):
    z = z_ref[...]
    m = m_ref[0] + m_ref[1]
    x = jnp.concatenate([z[:, 0:13], m[:, 0:13]], axis=1)
    h = jnp.tanh(
        jnp.dot(x, w1_ref[...], preferred_element_type=jnp.float32) + b1_ref[...])
    out_ref[...] = (
        jnp.dot(h, w2_ref[...], preferred_element_type=jnp.float32) + b2_ref[...])


def _node_call(z_pad, m_parts, w1, b1, w2, b2):
    grid = _BN // _BLK_N
    return pl.pallas_call(
        _node_body,
        grid=(grid,),
        in_specs=[
            pl.BlockSpec((_BLK_N, _FP), lambda i: (i, 0)),
            pl.BlockSpec((_NC, _BLK_N, _FP), lambda i: (0, i, 0)),
            pl.BlockSpec((2 * _F, _H), lambda i: (0, 0)),
            pl.BlockSpec((1, _H), lambda i: (0, 0)),
            pl.BlockSpec((_H, _F), lambda i: (0, 0)),
            pl.BlockSpec((1, _F), lambda i: (0, 0)),
        ],
        out_specs=pl.BlockSpec((_BLK_N, _F), lambda i: (i, 0)),
        out_shape=jax.ShapeDtypeStruct((_BN, _F), jnp.float32),
        compiler_params=pltpu.CompilerParams(
            dimension_semantics=("arbitrary",)),
    )(z_pad, m_parts, w1, b1, w2, b2)


def kernel(z_h, edge_index_h_h, We1, be1, We2, be2, Ww1, bw1, Ww2, bw2,
           Wn1, bn1, Wn2, bn2):
    z_flat = z_h.reshape(_BN, _F)
    z_pad = jnp.pad(z_flat, ((0, 0), (0, _FP - _F)))

    src = edge_index_h_h[0].astype(jnp.int32)
    tgt = edge_index_h_h[1].astype(jnp.int32)
    src2 = jnp.pad(src, (0, _E_PAD - _E)).reshape(_NW * _NBATCH, _EB)
    tgt2 = jnp.pad(tgt, (0, _E_PAD - _E)).reshape(_NW * _NBATCH, _EB)
    teff = _teff_call(src2, tgt2)

    zs, zt = _gather_call()(z_pad, src2, teff)

    w1c = jnp.concatenate([We1, Ww1], axis=1)
    b1c = jnp.concatenate([be1, bw1])[None, :]
    w64 = jnp.zeros((4 * _FP, 2 * _H), jnp.float32)
    w64 = w64.at[0:_F].set(w1c[0:13])
    w64 = w64.at[0:3].add(w1c[26:29])
    w64 = w64.at[_FP:_FP + _F].set(w1c[13:26])
    w64 = w64.at[_FP:_FP + 3].add(-w1c[26:29])
    w64 = w64.at[2 * _FP + 3:2 * _FP + 6].set(w1c[30:33])
    w64 = w64.at[3 * _FP + 0:3 * _FP + 3].set(
        jnp.broadcast_to(w1c[29], (3, 2 * _H)))
    w64 = w64.at[3 * _FP + 6].set(w1c[33])
    w2c = jnp.zeros((2 * _H, _FP), jnp.float32)
    w2c = w2c.at[0:_H, 0:_F].set(We2)
    w2c = w2c.at[_H:2 * _H, _F:_F + 1].set(Ww2)
    b2c = jnp.zeros((_FP,), jnp.float32)
    b2c = b2c.at[0:_F].set(be2)
    b2c = b2c.at[_F].set(bw2[0])
    b2c = b2c[None, :]
    m_mat = jnp.zeros((_FP, _FP), jnp.float32).at[3:6].set(1.0)
    esel = jnp.zeros((_FP, _FP), jnp.float32).at[_F].set(1.0)
    p1 = jnp.zeros((_FP, _FP), jnp.float32)
    p1 = p1.at[4, 3].set(1.0).at[5, 4].set(1.0).at[3, 5].set(1.0)
    p2 = jnp.zeros((_FP, _FP), jnp.float32)
    p2 = p2.at[5, 3].set(1.0).at[3, 4].set(1.0).at[4, 5].set(1.0)
    rot_big = jnp.zeros((2 * _FP, 4 * _FP), jnp.float32)
    rot_big = rot_big.at[0:_FP, 0:_FP].set(p1)
    rot_big = rot_big.at[0:_FP, _FP:2 * _FP].set(p2)
    rot_big = rot_big.at[_FP:2 * _FP, 2 * _FP:3 * _FP].set(p2)
    rot_big = rot_big.at[_FP:2 * _FP, 3 * _FP:4 * _FP].set(p1)
    eye = jnp.eye(_FP, dtype=jnp.float32)
    sub_big = jnp.concatenate([eye, -eye], axis=0)

    wm_p = _edge_call(zs.reshape(_E_PAD // 8, 8 * _FP),
                      zt.reshape(_E_PAD // 8, 8 * _FP),
                      w64, b1c, w2c, b2c, m_mat, esel, rot_big, sub_big)
    wm = wm_p.reshape(_E_PAD, _FP)

    zeros_acc = jnp.zeros((_BN, _FP), jnp.float32)
    m_parts = _scatter_call()(wm, teff, zeros_acc)

    delta = _node_call(z_pad, m_parts.reshape(_NC, _BN, _FP),
                       Wn1, bn1[None, :], Wn2, bn2[None, :])
    return delta.reshape(_B, _N, _F)


# speedup vs baseline: 15.8125x; 1.1700x over previous
import functools

import jax
import jax.numpy as jnp
from jax import lax
from jax.experimental import pallas as pl
from jax.experimental.pallas import tpu as pltpu
from jax.experimental.pallas import tpu_sc as plsc

_B, _N, _F, _E, _H = 4, 2500, 13, 320000, 128
_BN = _B * _N
_FP = 16

_NC, _NS = 2, 16
_NW = _NC * _NS
_EB = 128
_NBATCH = 80
_GRP = 4
_E_PAD = _NW * _NBATCH * _EB
_NBT = _NW * _NBATCH
_GNB0, _GNB1 = 80, 80
_SNB0, _SNB1 = 88, 72
_ROWS_PER_TILE = _BN // _NS

_BLK_E = 4096
_BLK_N = 2000

@functools.lru_cache(maxsize=None)
def _sc_mesh():
    return plsc.VectorSubcoreMesh(
        core_axis_name="c", subcore_axis_name="s",
        num_cores=_NC, num_subcores=_NS)


def _teff_body(src_ref, tgt_ref, out_ref):
    s = src_ref[...]
    t = tgt_ref[...]
    out_ref[...] = s - lax.rem(s, _N) + lax.rem(t, _N)


def _teff_call(src_p, tgt_p):
    n_rows = _NW * _NBATCH
    return pl.pallas_call(
        _teff_body,
        grid=(1,),
        in_specs=[
            pl.BlockSpec((n_rows, _EB), lambda i: (0, 0)),
            pl.BlockSpec((n_rows, _EB), lambda i: (0, 0)),
        ],
        out_specs=pl.BlockSpec((n_rows, _EB), lambda i: (0, 0)),
        out_shape=jax.ShapeDtypeStruct((n_rows, _EB), jnp.int32),
    )(src_p, tgt_p)


def _gather_pipe(z_hbm, src_hbm, teff_hbm, zs_out, zt_out,
                 idx_s, idx_t, rows_s, rows_t, sems_g, sems_o, b0, nb):
    pltpu.sync_copy(src_hbm.at[pl.ds(b0, nb)], idx_s.at[pl.ds(0, nb)])
    pltpu.sync_copy(teff_hbm.at[pl.ds(b0, nb)], idx_t.at[pl.ds(0, nb)])
    base = b0 * _EB

    grows = _GRP * _EB
    rows_bufs = (rows_s, rows_t)
    outs = (zs_out, zt_out)

    def _fire(g, p):
        descs = []
        for j in range(_GRP):
            b = g * _GRP + j
            for idx, rows, sem in ((idx_s, rows_bufs[0], sems_g[p][0]),
                                   (idx_t, rows_bufs[1], sems_g[p][1])):
                descs.append(pltpu.async_copy(
                    z_hbm.at[idx.at[b]],
                    rows.at[pl.ds((p * _GRP + j) * _EB, _EB)], sem))
        return descs

    def _drain_out(p):
        for rows, out in zip(rows_bufs, outs):
            pltpu.make_async_copy(
                rows.at[pl.ds(p * grows, grows)],
                out.at[pl.ds(base, grows)], sems_o[p]).wait()

    def _pair(i, carry):
        descs = []
        for p in (0, 1):
            @pl.when(i > 0)
            def _():
                _drain_out(p)
            descs.append(_fire(2 * i + p, p))
        for p in (0, 1):
            for d in descs[p]:
                d.wait()
            row0 = base + (2 * i + p) * grows
            for rows, out in zip(rows_bufs, outs):
                pltpu.async_copy(rows.at[pl.ds(p * grows, grows)],
                                 out.at[pl.ds(row0, grows)], sems_o[p])
        return carry
    lax.fori_loop(0, nb // (2 * _GRP), _pair, 0)
    _drain_out(0)
    _drain_out(1)


def _gather_body(z_hbm, src_hbm, teff_hbm, zs_out, zt_out,
                 idx_s, idx_t, rows_s, rows_t, z_sp,
                 sem_s0, sem_t0, sem_s1, sem_t1, sem_o0, sem_o1):
    c = lax.axis_index("c")
    s = lax.axis_index("s")
    sems_g = ((sem_s0, sem_t0), (sem_s1, sem_t1))
    sems_o = (sem_o0, sem_o1)

    r0 = s * _ROWS_PER_TILE
    pltpu.sync_copy(z_hbm.at[pl.ds(r0, _ROWS_PER_TILE)],
                    z_sp.at[pl.ds(r0, _ROWS_PER_TILE)])
    plsc.subcore_barrier()

    @pl.when(c == 0)
    def _():
        _gather_pipe(z_sp, src_hbm, teff_hbm, zs_out, zt_out,
                     idx_s, idx_t, rows_s, rows_t, sems_g, sems_o,
                     s * _GNB0, _GNB0)

    @pl.when(c == 1)
    def _():
        _gather_pipe(z_sp, src_hbm, teff_hbm, zs_out, zt_out,
                     idx_s, idx_t, rows_s, rows_t, sems_g, sems_o,
                     _NS * _GNB0 + s * _GNB1, _GNB1)


@functools.lru_cache(maxsize=None)
def _gather_call():
    return pl.kernel(
        _gather_body,
        out_type=(
            jax.ShapeDtypeStruct((_E_PAD, _FP), jnp.float32),
            jax.ShapeDtypeStruct((_E_PAD, _FP), jnp.float32),
        ),
        mesh=_sc_mesh(),
        scratch_types=(
            pltpu.VMEM((_GNB0, _EB), jnp.int32),
            pltpu.VMEM((_GNB0, _EB), jnp.int32),
            pltpu.VMEM((2 * _GRP * _EB, _FP), jnp.float32),
            pltpu.VMEM((2 * _GRP * _EB, _FP), jnp.float32),
            pltpu.VMEM_SHARED((_BN, _FP), jnp.float32),
            pltpu.SemaphoreType.DMA,
            pltpu.SemaphoreType.DMA,
            pltpu.SemaphoreType.DMA,
            pltpu.SemaphoreType.DMA,
            pltpu.SemaphoreType.DMA,
            pltpu.SemaphoreType.DMA,
        ),
        compiler_params=pltpu.CompilerParams(use_tc_tiling_on_sc=False),
    )


def _scatter_pipe(wm_hbm, teff_hbm, idx_t, rows, acc, sem, b0, nb):
    pltpu.sync_copy(teff_hbm.at[pl.ds(b0, nb)], idx_t.at[pl.ds(0, nb)])

    def _gbody(g, carry):
        pltpu.sync_copy(
            wm_hbm.at[pl.ds(b0 * _EB + g * (_GRP * _EB), _GRP * _EB)], rows)
        descs = []
        for j in range(_GRP):
            descs.append(pltpu.async_copy(
                rows.at[pl.ds(j * _EB, _EB)], acc.at[idx_t.at[g * _GRP + j]],
                sem, add=True))
        for d in descs:
            d.wait()
        return carry
    lax.fori_loop(0, nb // _GRP, _gbody, 0)


def _scatter_body(wm_hbm, teff_hbm, zeros_hbm, out_hbm, idx_t, rows, acc, sem):
    c = lax.axis_index("c")
    s = lax.axis_index("s")
    r0 = s * _ROWS_PER_TILE

    pltpu.sync_copy(zeros_hbm.at[pl.ds(r0, _ROWS_PER_TILE)],
                    acc.at[pl.ds(r0, _ROWS_PER_TILE)])
    plsc.subcore_barrier()

    @pl.when(c == 0)
    def _():
        _scatter_pipe(wm_hbm, teff_hbm, idx_t, rows, acc, sem,
                      s * _SNB0, _SNB0)

    @pl.when(c == 1)
    def _():
        _scatter_pipe(wm_hbm, teff_hbm, idx_t, rows, acc, sem,
                      _NS * _SNB0 + s * _SNB1, _SNB1)

    plsc.subcore_barrier()
    pltpu.sync_copy(acc.at[pl.ds(r0, _ROWS_PER_TILE)],
                    out_hbm.at[pl.ds(c * _BN + r0, _ROWS_PER_TILE)])


@functools.lru_cache(maxsize=None)
def _scatter_call():
    return pl.kernel(
        _scatter_body,
        out_type=jax.ShapeDtypeStruct((_NC * _BN, _FP), jnp.float32),
        mesh=_sc_mesh(),
        scratch_types=(
            pltpu.VMEM((_SNB0, _EB), jnp.int32),
            pltpu.VMEM((_GRP * _EB, _FP), jnp.float32),
            pltpu.VMEM_SHARED((_BN, _FP), jnp.float32),
            pltpu.SemaphoreType.DMA,
        ),
        compiler_params=pltpu.CompilerParams(use_tc_tiling_on_sc=False),
    )


def _edge_body(zs_ref, zt_ref, w1_ref, b1_ref, w2_ref, b2_ref, m_ref,
               esel_ref, p1_ref, p2_ref, wm_ref):
    zp = zs_ref[...]
    zq = zt_ref[...]
    zs = jnp.concatenate([zp[:, k * _FP:(k + 1) * _FP] for k in range(8)],
                         axis=0)
    zt = jnp.concatenate([zq[:, k * _FP:(k + 1) * _FP] for k in range(8)],
                         axis=0)
    col = lax.broadcasted_iota(jnp.int32, zs.shape, 1)
    d = zs - zt
    dsq = d * d
    zz = jnp.concatenate([zs, zt], axis=1)
    rots = jnp.dot(zz, p1_ref[...], preferred_element_type=jnp.float32)
    prod = rots[:, 0:2 * _FP] * rots[:, 2 * _FP:4 * _FP]
    cross = jnp.dot(prod, p2_ref[...], preferred_element_type=jnp.float32)
    csq = cross * cross
    s2 = jnp.dot(csq, m_ref[...], preferred_element_type=jnp.float32)
    absc = jnp.sqrt(s2)
    quad = jnp.where(col == 6, absc, jnp.where(col < 3, dsq, 0.0))
    x = jnp.concatenate([zz, cross, quad], axis=1)
    h = jnp.tanh(
        jnp.dot(x, w1_ref[...], preferred_element_type=jnp.float32) + b1_ref[...])
    y = jnp.dot(h, w2_ref[...], preferred_element_type=jnp.float32) + b2_ref[...]
    w = jax.nn.sigmoid(
        jnp.dot(y, esel_ref[...], preferred_element_type=jnp.float32))
    p_row = lax.broadcasted_iota(jnp.int32, y.shape, 0)
    rows8 = _BLK_E // 8
    e_loc = 8 * (p_row % rows8) + p_row // rows8
    row = e_loc + pl.program_id(0) * _BLK_E
    wm = jnp.where((col < _F) & (row < _E), y * w, 0.0)
    wm_ref[...] = jnp.concatenate(
        [wm[k * rows8:(k + 1) * rows8, :] for k in range(8)], axis=1)


def _edge_call(zs, zt, w1, b1, w2, b2, m_mat, esel, p1, p2):
    grid = _E_PAD // _BLK_E
    return pl.pallas_call(
        _edge_body,
        grid=(grid,),
        in_specs=[
            pl.BlockSpec((_BLK_E // 8, 8 * _FP), lambda i: (i, 0)),
            pl.BlockSpec((_BLK_E // 8, 8 * _FP), lambda i: (i, 0)),
            pl.BlockSpec((4 * _FP, 2 * _H), lambda i: (0, 0)),
            pl.BlockSpec((1, 2 * _H), lambda i: (0, 0)),
            pl.BlockSpec((2 * _H, _FP), lambda i: (0, 0)),
            pl.BlockSpec((1, _FP), lambda i: (0, 0)),
            pl.BlockSpec((_FP, _FP), lambda i: (0, 0)),
            pl.BlockSpec((_FP, _FP), lambda i: (0, 0)),
            pl.BlockSpec((2 * _FP, 4 * _FP), lambda i: (0, 0)),
            pl.BlockSpec((2 * _FP, _FP), lambda i: (0, 0)),
        ],
        out_specs=pl.BlockSpec((_BLK_E // 8, 8 * _FP), lambda i: (i, 0)),
        out_shape=jax.ShapeDtypeStruct((_E_PAD // 8, 8 * _FP), jnp.float32),
        compiler_params=pltpu.CompilerParams(
            dimension_semantics=("arbitrary",)),
    )(zs, zt, w1, b1, w2, b2, m_mat, esel, p1, p2)


def _node_body(z_ref, m_ref, w1_ref, b1_ref, w2_ref, b2_ref, out_ref):
    z = z_ref[...]
    m = m_ref[0] + m_ref[1]
    x = jnp.concatenate([z[:, 0:13], m[:, 0:13]], axis=1)
    h = jnp.tanh(
        jnp.dot(x, w1_ref[...], preferred_element_type=jnp.float32) + b1_ref[...])
    out_ref[...] = (
        jnp.dot(h, w2_ref[...], preferred_element_type=jnp.float32) + b2_ref[...])


def _node_call(z_pad, m_parts, w1, b1, w2, b2):
    grid = _BN // _BLK_N
    return pl.pallas_call(
        _node_body,
        grid=(grid,),
        in_specs=[
            pl.BlockSpec((_BLK_N, _FP), lambda i: (i, 0)),
            pl.BlockSpec((_NC, _BLK_N, _FP), lambda i: (0, i, 0)),
            pl.BlockSpec((2 * _F, _H), lambda i: (0, 0)),
            pl.BlockSpec((1, _H), lambda i: (0, 0)),
            pl.BlockSpec((_H, _F), lambda i: (0, 0)),
            pl.BlockSpec((1, _F), lambda i: (0, 0)),
        ],
        out_specs=pl.BlockSpec((_BLK_N, _F), lambda i: (i, 0)),
        out_shape=jax.ShapeDtypeStruct((_BN, _F), jnp.float32),
        compiler_params=pltpu.CompilerParams(
            dimension_semantics=("arbitrary",)),
    )(z_pad, m_parts, w1, b1, w2, b2)


def kernel(z_h, edge_index_h_h, We1, be1, We2, be2, Ww1, bw1, Ww2, bw2,
           Wn1, bn1, Wn2, bn2):
    z_flat = z_h.reshape(_BN, _F)
    z_pad = jnp.pad(z_flat, ((0, 0), (0, _FP - _F)))

    src = edge_index_h_h[0].astype(jnp.int32)
    tgt = edge_index_h_h[1].astype(jnp.int32)
    src2 = jnp.pad(src, (0, _E_PAD - _E)).reshape(_NW * _NBATCH, _EB)
    tgt2 = jnp.pad(tgt, (0, _E_PAD - _E)).reshape(_NW * _NBATCH, _EB)
    teff = _teff_call(src2, tgt2)

    zs, zt = _gather_call()(z_pad, src2, teff)

    w1c = jnp.concatenate([We1, Ww1], axis=1)
    b1c = jnp.concatenate([be1, bw1])[None, :]
    w64 = jnp.zeros((4 * _FP, 2 * _H), jnp.float32)
    w64 = w64.at[0:_F].set(w1c[0:13])
    w64 = w64.at[0:3].add(w1c[26:29])
    w64 = w64.at[_FP:_FP + _F].set(w1c[13:26])
    w64 = w64.at[_FP:_FP + 3].add(-w1c[26:29])
    w64 = w64.at[2 * _FP + 3:2 * _FP + 6].set(w1c[30:33])
    w64 = w64.at[3 * _FP + 0:3 * _FP + 3].set(
        jnp.broadcast_to(w1c[29], (3, 2 * _H)))
    w64 = w64.at[3 * _FP + 6].set(w1c[33])
    w2c = jnp.zeros((2 * _H, _FP), jnp.float32)
    w2c = w2c.at[0:_H, 0:_F].set(We2)
    w2c = w2c.at[_H:2 * _H, _F:_F + 1].set(Ww2)
    b2c = jnp.zeros((_FP,), jnp.float32)
    b2c = b2c.at[0:_F].set(be2)
    b2c = b2c.at[_F].set(bw2[0])
    b2c = b2c[None, :]
    m_mat = jnp.zeros((_FP, _FP), jnp.float32).at[3:6].set(1.0)
    esel = jnp.zeros((_FP, _FP), jnp.float32).at[_F].set(1.0)
    p1 = jnp.zeros((_FP, _FP), jnp.float32)
    p1 = p1.at[4, 3].set(1.0).at[5, 4].set(1.0).at[3, 5].set(1.0)
    p2 = jnp.zeros((_FP, _FP), jnp.float32)
    p2 = p2.at[5, 3].set(1.0).at[3, 4].set(1.0).at[4, 5].set(1.0)
    rot_big = jnp.zeros((2 * _FP, 4 * _FP), jnp.float32)
    rot_big = rot_big.at[0:_FP, 0:_FP].set(p1)
    rot_big = rot_big.at[0:_FP, _FP:2 * _FP].set(p2)
    rot_big = rot_big.at[_FP:2 * _FP, 2 * _FP:3 * _FP].set(p2)
    rot_big = rot_big.at[_FP:2 * _FP, 3 * _FP:4 * _FP].set(p1)
    eye = jnp.eye(_FP, dtype=jnp.float32)
    sub_big = jnp.concatenate([eye, -eye], axis=0)

    wm_p = _edge_call(zs.reshape(_E_PAD // 8, 8 * _FP),
                      zt.reshape(_E_PAD // 8, 8 * _FP),
                      w64, b1c, w2c, b2c, m_mat, esel, rot_big, sub_big)
    wm = wm_p.reshape(_E_PAD, _FP)

    zeros_acc = jnp.zeros((_BN, _FP), jnp.float32)
    m_parts = _scatter_call()(wm, teff, zeros_acc)

    delta = _node_call(z_pad, m_parts.reshape(_NC, _BN, _FP),
                       Wn1, bn1[None, :], Wn2, bn2[None, :])
    return delta.reshape(_B, _N, _F)
